# baseline (speedup 1.0000x reference)
import functools

import jax
import jax.numpy as jnp
from jax import lax
from jax.experimental import pallas as pl
from jax.experimental.pallas import tpu as pltpu
from jax.experimental.pallas import tpu_sc as plsc

EPS = 1e-12

_NUM_CORES = 2
_NUM_SUBCORES = 16
_NW = _NUM_CORES * _NUM_SUBCORES
_CHUNK = 64


@functools.lru_cache(maxsize=None)
def _make_sc_gather(n_tokens: int, hidden: int):
    assert n_tokens % (_NW * _CHUNK) == 0
    per_w = n_tokens // _NW
    n_chunks = per_w // _CHUNK
    mesh = plsc.VectorSubcoreMesh(core_axis_name="c", subcore_axis_name="s")

    @functools.partial(
        pl.kernel,
        out_type=jax.ShapeDtypeStruct((n_tokens, hidden), jnp.float32),
        mesh=mesh,
        scratch_types=[
            pltpu.VMEM((per_w,), jnp.int32),
            pltpu.VMEM((_CHUNK, hidden), jnp.float32),
            pltpu.VMEM((_CHUNK, hidden), jnp.float32),
            pltpu.SemaphoreType.DMA,
            pltpu.SemaphoreType.DMA,
        ],
    )
    def gather(word_hbm, ids_hbm, out_hbm, idx_v, rows0, rows1, sem0, sem1):
        c = lax.axis_index("c")
        s = lax.axis_index("s")
        wid = s * _NUM_CORES + c
        base = wid * per_w
        pltpu.sync_copy(ids_hbm.at[pl.ds(base, per_w)], idx_v)
        bufs = (rows0, rows1)
        sems = (sem0, sem1)
        copies = [None] * n_chunks
        copies[0] = pltpu.async_copy(
            word_hbm.at[idx_v.at[pl.ds(0, _CHUNK)]], bufs[0], sems[0])
        for j in range(n_chunks):
            if j + 1 < n_chunks:
                copies[j + 1] = pltpu.async_copy(
                    word_hbm.at[idx_v.at[pl.ds((j + 1) * _CHUNK, _CHUNK)]],
                    bufs[(j + 1) % 2], sems[(j + 1) % 2])
            copies[j].wait()
            pltpu.sync_copy(
                bufs[j % 2], out_hbm.at[pl.ds(base + j * _CHUNK, _CHUNK)])

    return gather


def _ln_body(emb_ref, pos_ref, pos2_ref, w_ref, b_ref, out_ref):
    x = pos_ref[...] * emb_ref[...] + pos2_ref[...]
    mean = jnp.mean(x, axis=-1, keepdims=True)
    xc = x - mean
    var = jnp.mean(xc * xc, axis=-1, keepdims=True)
    y = xc * lax.rsqrt(var + EPS)
    out_ref[...] = y * w_ref[...] + b_ref[...]


def _ln_call(emb, pos, pos2, w, b, block_tokens: int):
    n, hidden = emb.shape
    s_len = pos.shape[0]
    assert n % block_tokens == 0 and s_len % block_tokens == 0
    s_blocks = s_len // block_tokens
    batch = n // s_len
    return pl.pallas_call(
        _ln_body,
        grid=(s_blocks, batch),
        in_specs=[
            pl.BlockSpec((block_tokens, hidden), lambda j, bi: (bi * s_blocks + j, 0)),
            pl.BlockSpec((block_tokens, hidden), lambda j, bi: (j, 0)),
            pl.BlockSpec((block_tokens, hidden), lambda j, bi: (j, 0)),
            pl.BlockSpec((1, hidden), lambda j, bi: (0, 0)),
            pl.BlockSpec((1, hidden), lambda j, bi: (0, 0)),
        ],
        out_specs=pl.BlockSpec(
            (block_tokens, hidden), lambda j, bi: (bi * s_blocks + j, 0)),
        out_shape=jax.ShapeDtypeStruct((n, hidden), jnp.float32),
    )(emb, pos, pos2, w, b)


_PARTS = 2


def kernel(input_ids, word_emb, pos_emb, pos_emb2, ln_weight, ln_bias):
    b, s = input_ids.shape
    hidden = word_emb.shape[1]
    n = b * s
    ids = input_ids.reshape(n).astype(jnp.int32)
    pos = pos_emb[:s]
    pos2 = pos_emb2[:s]
    w = ln_weight.reshape(1, hidden)
    bb = ln_bias.reshape(1, hidden)
    n_p = n // _PARTS
    assert n_p % s == 0
    emb_parts = []
    ln_parts = []
    for p in range(_PARTS):
        ids_p = lax.slice_in_dim(ids, p * n_p, (p + 1) * n_p)
        emb_p = _make_sc_gather(n_p, hidden)(word_emb, ids_p)
        emb_parts.append(emb_p)
        ln_parts.append(_ln_call(emb_p, pos, pos2, w, bb, block_tokens=2048))
    embeddings = jnp.concatenate(ln_parts, axis=0)
    inputs_embeds = jnp.concatenate(emb_parts, axis=0)
    return (embeddings.reshape(b, s, hidden),
            inputs_embeds.reshape(b, s, hidden))


# speedup vs baseline: 1.6981x; 1.6981x over previous
import functools

import jax
import jax.numpy as jnp
from jax import lax
from jax.experimental import pallas as pl
from jax.experimental.pallas import tpu as pltpu
from jax.experimental.pallas import tpu_sc as plsc

EPS = 1e-12

_NUM_CORES = 2
_NUM_SUBCORES = 16
_NW = _NUM_CORES * _NUM_SUBCORES
_CHUNK = 32
_NBUF = 4


@functools.lru_cache(maxsize=None)
def _make_sc_gather(n_tokens: int, hidden: int):
    assert n_tokens % (_NW * _CHUNK) == 0
    per_w = n_tokens // _NW
    n_chunks = per_w // _CHUNK
    mesh = plsc.VectorSubcoreMesh(core_axis_name="c", subcore_axis_name="s")

    @functools.partial(
        pl.kernel,
        out_type=jax.ShapeDtypeStruct((n_tokens, hidden), jnp.float32),
        mesh=mesh,
        scratch_types=(
            [pltpu.VMEM((per_w,), jnp.int32)]
            + [pltpu.VMEM((_CHUNK, hidden), jnp.float32)] * _NBUF
            + [pltpu.SemaphoreType.DMA] * _NBUF
            + [pltpu.SemaphoreType.DMA] * _NBUF
        ),
    )
    def gather(word_hbm, ids_hbm, out_hbm, idx_v, *bufs_sems):
        bufs = bufs_sems[:_NBUF]
        gsems = bufs_sems[_NBUF:2 * _NBUF]
        ssems = bufs_sems[2 * _NBUF:]
        c = lax.axis_index("c")
        s = lax.axis_index("s")
        wid = s * _NUM_CORES + c
        base = wid * per_w
        pltpu.sync_copy(ids_hbm.at[pl.ds(base, per_w)], idx_v)
        gathers = [None] * n_chunks
        stores = [None] * n_chunks
        store_waited = [False] * n_chunks
        depth = min(_NBUF - 1, n_chunks)
        for j in range(depth):
            gathers[j] = pltpu.async_copy(
                word_hbm.at[idx_v.at[pl.ds(j * _CHUNK, _CHUNK)]],
                bufs[j % _NBUF], gsems[j % _NBUF])
        for j in range(n_chunks):
            gathers[j].wait()
            stores[j] = pltpu.async_copy(
                bufs[j % _NBUF], out_hbm.at[pl.ds(base + j * _CHUNK, _CHUNK)],
                ssems[j % _NBUF])
            nxt = j + depth
            if nxt < n_chunks:
                prev = nxt - _NBUF
                if prev >= 0:
                    stores[prev].wait()
                    store_waited[prev] = True
                gathers[nxt] = pltpu.async_copy(
                    word_hbm.at[idx_v.at[pl.ds(nxt * _CHUNK, _CHUNK)]],
                    bufs[nxt % _NBUF], gsems[nxt % _NBUF])
        for j in range(n_chunks):
            if not store_waited[j]:
                stores[j].wait()

    return gather


def _ln_body(emb_ref, pos_ref, pos2_ref, w_ref, b_ref, out_ref):
    x = pos_ref[...] * emb_ref[...] + pos2_ref[...]
    mean = jnp.mean(x, axis=-1, keepdims=True)
    xc = x - mean
    var = jnp.mean(xc * xc, axis=-1, keepdims=True)
    y = xc * lax.rsqrt(var + EPS)
    out_ref[...] = y * w_ref[...] + b_ref[...]


def _ln_call(emb, pos, pos2, w, b, block_tokens: int):
    n, hidden = emb.shape
    s_len = pos.shape[0]
    assert n % block_tokens == 0 and s_len % block_tokens == 0
    s_blocks = s_len // block_tokens
    batch = n // s_len
    return pl.pallas_call(
        _ln_body,
        grid=(s_blocks, batch),
        in_specs=[
            pl.BlockSpec((block_tokens, hidden), lambda j, bi: (bi * s_blocks + j, 0)),
            pl.BlockSpec((block_tokens, hidden), lambda j, bi: (j, 0)),
            pl.BlockSpec((block_tokens, hidden), lambda j, bi: (j, 0)),
            pl.BlockSpec((1, hidden), lambda j, bi: (0, 0)),
            pl.BlockSpec((1, hidden), lambda j, bi: (0, 0)),
        ],
        out_specs=pl.BlockSpec(
            (block_tokens, hidden), lambda j, bi: (bi * s_blocks + j, 0)),
        out_shape=jax.ShapeDtypeStruct((n, hidden), jnp.float32),
    )(emb, pos, pos2, w, b)


def kernel(input_ids, word_emb, pos_emb, pos_emb2, ln_weight, ln_bias):
    b, s = input_ids.shape
    hidden = word_emb.shape[1]
    n = b * s
    ids = input_ids.reshape(n).astype(jnp.int32)
    inputs_embeds = _make_sc_gather(n, hidden)(word_emb, ids)
    pos = pos_emb[:s]
    pos2 = pos_emb2[:s]
    embeddings = _ln_call(
        inputs_embeds, pos, pos2,
        ln_weight.reshape(1, hidden), ln_bias.reshape(1, hidden),
        block_tokens=2048)
    return (embeddings.reshape(b, s, hidden),
            inputs_embeds.reshape(b, s, hidden))
